# baseline (speedup 1.0000x reference)
import functools

import jax
import jax.numpy as jnp
from jax import lax
from jax.experimental import pallas as pl
from jax.experimental.pallas import tpu as pltpu
from jax.experimental.pallas import tpu_sc as plsc

_VOCAB = 100000
_V = 50
_B = 1024
_D = 64
_K = 20
_NC = 2
_NS = 16
_NW = _NC * _NS
_ROWS = (_V * _B) // _NW
_C = 5
_CK = _C * _K
_NCHUNK = _ROWS // _C
_GRP = 8
_GROWS = _GRP * _C
_NGRP = _NCHUNK // _GRP
_NH = 2
_NBUF = 4

_U = 16
_NUNIT = _VOCAB // _U


def _tree_sum(terms):
    while len(terms) > 1:
        nxt = [terms[i] + terms[i + 1] for i in range(0, len(terms) - 1, 2)]
        if len(terms) % 2:
            nxt.append(terms[-1])
        terms = nxt
    return terms[0]


def _sc_body(table, idx, pos, out, tb, idx_v, pos_v, rows_v, outc_v,
             cin_v, cout_v,
             gsem0, gsem1, gsem2, gsem3, osem, csem0, csem1, wsem):
    cid = lax.axis_index("c")
    sid = lax.axis_index("s")
    wid = sid * _NC + cid
    base = wid * _ROWS
    gsems = (gsem0, gsem1, gsem2, gsem3)
    csems = (csem0, csem1)
    tbc = tb.at[cid]

    pltpu.sync_copy(idx.at[wid], idx_v)
    pltpu.sync_copy(pos, pos_v)

    kmax = (_NUNIT - 1 - 0) // _NS + 1

    def cast_issue(k, s):
        uu = sid + k * _NS
        off = pl.multiple_of(uu * _U, _U)
        pltpu.async_copy(table.at[pl.ds(off, _U)], cin_v.at[s], csems[s])

    def cast_wait(s):
        pltpu.make_async_copy(table.at[pl.ds(0, _U)], cin_v.at[s],
                              csems[s]).wait()

    def cast_unit(k, s):
        cast_wait(s)
        for r in range(_U):
            for h in range(_NH):
                a = cin_v[s, r, pl.ds(h * 32, 16)]
                b = cin_v[s, r, pl.ds(h * 32 + 16, 16)]
                cout_v[s, r, pl.ds(h * 32, 32)] = plsc.pack(
                    a, b, format=plsc.PackFormat.INTERLEAVED)
        uu = sid + k * _NS
        off = pl.multiple_of(uu * _U, _U)
        pltpu.async_copy(cout_v.at[s], tbc.at[pl.ds(off, _U)], wsem)

    def cast_wwait():
        pltpu.make_async_copy(cout_v.at[0], tbc.at[pl.ds(0, _U)], wsem).wait()

    cast_issue(0, 0)

    def cast_loop(k, carry):
        s = lax.rem(k, 2)
        @pl.when(sid + (k + 1) * _NS < _NUNIT)
        def _():
            @pl.when(s == 0)
            def _():
                cast_issue(k + 1, 1)

            @pl.when(s == 1)
            def _():
                cast_issue(k + 1, 0)

        @pl.when(s == 0)
        def _():
            cast_unit(k, 0)

        @pl.when(s == 1)
        def _():
            cast_unit(k, 1)

        @pl.when(k >= 2)
        def _():
            cast_wwait()

        return carry

    nk = (_NUNIT - sid - 1) // _NS + 1
    lax.fori_loop(0, nk, cast_loop, 0)
    cast_wwait()
    cast_wwait()
    plsc.subcore_barrier()

    def issue(c, s):
        pltpu.async_copy(tbc.at[idx_v.at[c]], rows_v.at[s], gsems[s])

    def gwait(c, s):
        pltpu.make_async_copy(tbc.at[idx_v.at[c]], rows_v.at[s],
                              gsems[s]).wait()

    for s in range(_NBUF - 1):
        issue(s, s)

    def group(g, carry):
        ob = lax.rem(g, 2)
        @pl.when(g >= 2)
        def _():
            pltpu.make_async_copy(
                outc_v.at[ob], out.at[pl.ds(base, _GROWS)], osem).wait()

        for cc in range(_GRP):
            c = g * _GRP + cc
            pre = c + _NBUF - 1

            @pl.when(pre < _NCHUNK)
            def _(pre=pre, s=(cc + _NBUF - 1) % _NBUF):
                issue(pre, s)

            s = cc % _NBUF
            gwait(c, s)

            def row(rr, carry3, cc=cc, c=c, s=s):
                r = base + c * _C + rr
                vv = lax.shift_right_logical(r, 10)
                rbase = rr * _K
                orow = cc * _C + rr
                for h in range(_NH):
                    sl = pl.ds(h * 32, 32)
                    acc = _tree_sum(
                        [rows_v[s, rbase + d, sl] for d in range(_K)])
                    a, b = plsc.unpack(acc, format=plsc.PackFormat.INTERLEAVED)
                    outc_v[ob, orow, pl.ds(h * 32, 16)] = (
                        a + pos_v[vv, pl.ds(h * 32, 16)])
                    outc_v[ob, orow, pl.ds(h * 32 + 16, 16)] = (
                        b + pos_v[vv, pl.ds(h * 32 + 16, 16)])
                return carry3

            lax.fori_loop(0, _C, row, 0)

        off = pl.multiple_of(base + g * _GROWS, _GROWS)
        pltpu.async_copy(outc_v.at[ob], out.at[pl.ds(off, _GROWS)], osem)
        return carry

    lax.fori_loop(0, _NGRP, group, 0)

    for _ in range(2):
        pltpu.make_async_copy(
            outc_v.at[0], out.at[pl.ds(base, _GROWS)], osem).wait()


_mesh = plsc.VectorSubcoreMesh(core_axis_name="c", subcore_axis_name="s")

_sc_call = functools.partial(
    pl.kernel,
    out_type=(
        jax.ShapeDtypeStruct((_V * _B, _D), jnp.float32),
        jax.ShapeDtypeStruct((_NC, _VOCAB, _D), jnp.bfloat16),
    ),
    mesh=_mesh,
    scratch_types=[
        pltpu.VMEM((_NCHUNK, _CK), jnp.int32),
        pltpu.VMEM((_V, _D), jnp.float32),
        pltpu.VMEM((_NBUF, _CK, _D), jnp.bfloat16),
        pltpu.VMEM((2, _GROWS, _D), jnp.float32),
        pltpu.VMEM((2, _U, _D), jnp.float32),
        pltpu.VMEM((2, _U, _D), jnp.bfloat16),
        pltpu.SemaphoreType.DMA,
        pltpu.SemaphoreType.DMA,
        pltpu.SemaphoreType.DMA,
        pltpu.SemaphoreType.DMA,
        pltpu.SemaphoreType.DMA,
        pltpu.SemaphoreType.DMA,
        pltpu.SemaphoreType.DMA,
        pltpu.SemaphoreType.DMA,
    ],
    compiler_params=pltpu.CompilerParams(use_tc_tiling_on_sc=False,
                                         needs_layout_passes=False),
)(_sc_body)


def kernel(x, emb_diag, emb_pos):
    x = x.astype(jnp.int32)
    xt = jnp.transpose(x, (2, 0, 1)).reshape(_V * _B, _K)
    idx = xt.reshape(_NW, _NCHUNK, _CK)
    out, _ = _sc_call(emb_diag, idx, emb_pos)
    return out.reshape(_V, _B, _D)


# speedup vs baseline: 1.2112x; 1.2112x over previous
import functools

import numpy as np

import jax
import jax.numpy as jnp
from jax import lax
from jax.experimental import pallas as pl
from jax.experimental.pallas import tpu as pltpu
from jax.experimental.pallas import tpu_sc as plsc

_VOCAB = 100000
_V = 50
_B = 1024
_D = 64
_K = 20
_NC = 2
_NS = 16
_NW = _NC * _NS
_BPW = _B // _NW
_NH = 2

_OIDX = np.arange(_V, dtype=np.int32)[None, :] * _B + \
    np.arange(_B, dtype=np.int32)[:, None]


def _tree_sum(terms):
    while len(terms) > 1:
        nxt = [terms[i] + terms[i + 1] for i in range(0, len(terms) - 1, 2)]
        if len(terms) % 2:
            nxt.append(terms[-1])
        terms = nxt
    return terms[0]


def _sc_body(table, xr, pos, oidx, out, xb_v, oidx_v, pos_v, rows_v, acc_v,
             gsem0, gsem1, ssem):
    wid = lax.axis_index("s") * _NC + lax.axis_index("c")
    bbase = wid * _BPW
    gsems = (gsem0, gsem1)

    pltpu.sync_copy(xr.at[pl.ds(bbase, _BPW)], xb_v)
    pltpu.sync_copy(oidx.at[pl.ds(bbase, _BPW)], oidx_v)
    pltpu.sync_copy(pos, pos_v)

    def issue(bb, s):
        for d in range(_K):
            pltpu.async_copy(table.at[xb_v.at[bb, d]], rows_v.at[s, d],
                             gsems[s])

    def gwait(s):
        for d in range(_K):
            pltpu.make_async_copy(table.at[xb_v.at[0, d]], rows_v.at[s, d],
                                  gsems[s]).wait()

    def swait():
        pltpu.make_async_copy(acc_v.at[0], out.at[oidx_v.at[0]], ssem).wait()

    issue(0, 0)

    def pair(t, carry):
        for s in range(2):
            bb = 2 * t + s

            @pl.when(bb + 1 < _BPW)
            def _(bb=bb, os=1 - s):
                issue(bb + 1, os)

            gwait(s)

            @pl.when(bb >= 2)
            def _():
                swait()

            def row(rr, carry3, s=s):
                for h in range(_NH):
                    sl = pl.ds(h * 32, 32)
                    acc = _tree_sum(
                        [rows_v[s, d, rr, sl] for d in range(_K)])
                    a, b = plsc.unpack(acc, format=plsc.PackFormat.INTERLEAVED)
                    acc_v[s, rr, pl.ds(h * 32, 16)] = (
                        a + pos_v[rr, pl.ds(h * 32, 16)])
                    acc_v[s, rr, pl.ds(h * 32 + 16, 16)] = (
                        b + pos_v[rr, pl.ds(h * 32 + 16, 16)])
                return carry3

            lax.fori_loop(0, _V, row, 0)
            pltpu.async_copy(acc_v.at[s], out.at[oidx_v.at[bb]], ssem)
        return carry

    lax.fori_loop(0, _BPW // 2, pair, 0)

    swait()
    swait()


_mesh = plsc.VectorSubcoreMesh(core_axis_name="c", subcore_axis_name="s")

_sc_call = functools.partial(
    pl.kernel,
    out_type=jax.ShapeDtypeStruct((_V * _B, _D), jnp.float32),
    mesh=_mesh,
    scratch_types=[
        pltpu.VMEM((_BPW, _K, _V), jnp.int32),
        pltpu.VMEM((_BPW, _V), jnp.int32),
        pltpu.VMEM((_V, _D), jnp.float32),
        pltpu.VMEM((2, _K, _V, _D), jnp.bfloat16),
        pltpu.VMEM((2, _V, _D), jnp.float32),
        pltpu.SemaphoreType.DMA,
        pltpu.SemaphoreType.DMA,
        pltpu.SemaphoreType.DMA,
    ],
    compiler_params=pltpu.CompilerParams(use_tc_tiling_on_sc=False,
                                         needs_layout_passes=False),
)(_sc_body)


def kernel(x, emb_diag, emb_pos):
    x = x.astype(jnp.int32)
    tableb = (emb_diag.astype(jnp.bfloat16)
              .reshape(-1, 2, 2, 16).transpose(0, 1, 3, 2).reshape(-1, _D))
    out = _sc_call(tableb, x, emb_pos, jnp.asarray(_OIDX))
    return out.reshape(_V, _B, _D)


# speedup vs baseline: 1.5042x; 1.2419x over previous
import functools

import numpy as np

import jax
import jax.numpy as jnp
from jax import lax
from jax.experimental import pallas as pl
from jax.experimental.pallas import tpu as pltpu
from jax.experimental.pallas import tpu_sc as plsc

_V = 50
_B = 1024
_D = 64
_K = 20
_KH = _K // 2
_NC = 2
_NS = 16
_NW = _NC * _NS
_BPW = _B // _NW
_LANES = 16
_NJ = _D // _LANES

_OIDX = np.arange(_V, dtype=np.int32)[None, :] * _B + \
    np.arange(_B, dtype=np.int32)[:, None]


def _tree_sum(terms):
    while len(terms) > 1:
        nxt = [terms[i] + terms[i + 1] for i in range(0, len(terms) - 1, 2)]
        if len(terms) % 2:
            nxt.append(terms[-1])
        terms = nxt
    return terms[0]


def _sc_body(table, xr, pos, oidx, out, xb_v, oidx_v, pos_v, rows_v, acc_v,
             gsem0, gsem1, ssem):
    wid = lax.axis_index("s") * _NC + lax.axis_index("c")
    bbase = wid * _BPW
    gsems = (gsem0, gsem1)

    pltpu.sync_copy(xr.at[pl.ds(bbase, _BPW)], xb_v)
    pltpu.sync_copy(oidx.at[pl.ds(bbase, _BPW)], oidx_v)
    pltpu.sync_copy(pos, pos_v)

    def issue(bb, h):
        for d in range(_KH):
            pltpu.async_copy(table.at[xb_v.at[bb, h * _KH + d]],
                             rows_v.at[h, d], gsems[h])

    def gwait(h):
        for d in range(_KH):
            pltpu.make_async_copy(table.at[xb_v.at[0, d]], rows_v.at[h, d],
                                  gsems[h]).wait()

    def swait():
        pltpu.make_async_copy(acc_v.at[0], out.at[oidx_v.at[0]], ssem).wait()

    issue(0, 0)
    issue(0, 1)

    def pair(t, carry):
        for u in range(2):
            bb = 2 * t + u

            gwait(0)

            @pl.when(bb >= 2)
            def _():
                swait()

            def row_lo(rr, carry3, u=u):
                for j in range(_NJ):
                    sl = pl.ds(j * _LANES, _LANES)
                    acc = _tree_sum(
                        [pos_v[rr, sl]] +
                        [rows_v[0, d, rr, sl] for d in range(_KH)])
                    acc_v[u, rr, sl] = acc
                return carry3

            lax.fori_loop(0, _V, row_lo, 0)

            @pl.when(bb + 1 < _BPW)
            def _(bb=bb):
                issue(bb + 1, 0)

            gwait(1)

            def row_hi(rr, carry3, u=u):
                for j in range(_NJ):
                    sl = pl.ds(j * _LANES, _LANES)
                    acc = _tree_sum(
                        [acc_v[u, rr, sl]] +
                        [rows_v[1, d, rr, sl] for d in range(_KH)])
                    acc_v[u, rr, sl] = acc
                return carry3

            lax.fori_loop(0, _V, row_hi, 0)

            @pl.when(bb + 1 < _BPW)
            def _(bb=bb):
                issue(bb + 1, 1)

            pltpu.async_copy(acc_v.at[u], out.at[oidx_v.at[bb]], ssem)
        return carry

    lax.fori_loop(0, _BPW // 2, pair, 0)

    swait()
    swait()


_mesh = plsc.VectorSubcoreMesh(core_axis_name="c", subcore_axis_name="s")

_sc_call = functools.partial(
    pl.kernel,
    out_type=jax.ShapeDtypeStruct((_V * _B, _D), jnp.float32),
    mesh=_mesh,
    scratch_types=[
        pltpu.VMEM((_BPW, _K, _V), jnp.int32),
        pltpu.VMEM((_BPW, _V), jnp.int32),
        pltpu.VMEM((_V, _D), jnp.float32),
        pltpu.VMEM((2, _KH, _V, _D), jnp.float32),
        pltpu.VMEM((2, _V, _D), jnp.float32),
        pltpu.SemaphoreType.DMA,
        pltpu.SemaphoreType.DMA,
        pltpu.SemaphoreType.DMA,
    ],
    compiler_params=pltpu.CompilerParams(use_tc_tiling_on_sc=False),
)(_sc_body)


def kernel(x, emb_diag, emb_pos):
    x = x.astype(jnp.int32)
    out = _sc_call(emb_diag, x, emb_pos, jnp.asarray(_OIDX))
    return out.reshape(_V, _B, _D)
